# Initial kernel scaffold; baseline (speedup 1.0000x reference)
import jax
import jax.numpy as jnp
from jax.experimental import pallas as pl


def kernel(x, edge_index, edge_type, Wrel_0_0, Wroot_0_0, b_0_0, Wrel_0_1, Wroot_0_1, b_0_1, Wrel_1_0, Wroot_1_0, b_1_0, Wrel_1_1, Wroot_1_1, b_1_1, Wrel_2_0, Wroot_2_0, b_2_0, Wrel_2_1, Wroot_2_1, b_2_1, fc1_W, fc1_b, fc2_W, fc2_b):
    raise NotImplementedError("write your pallas kernel here")


# speedup vs baseline: 5.4378x; 5.4378x over previous
import functools

import jax
import jax.numpy as jnp
from jax import lax
from jax.experimental import pallas as pl
from jax.experimental.pallas import tpu as pltpu
from jax.experimental.pallas import tpu_sc as plsc

N = 10000
E = 320000
D = 128
H = 128
NUM_REL = 4
NCLS = 16
METAPATHS = ((0, 1), (2, 3), (1, 0))

NC = 2
NS = 16
NW = NC * NS
LANES = 16
CHUNK = E // NW
VPC = CHUNK // LANES
NPAD = 10240
TRASH = N
BLK = 128
SLAB = NPAD // NS

BROWS = 1000
GRID = N // BROWS


def _mesh():
    return plsc.VectorSubcoreMesh(core_axis_name="c", subcore_axis_name="s",
                                  num_cores=NC, num_subcores=NS)


_SC_PARAMS = pltpu.CompilerParams(needs_layout_passes=False,
                                  use_tc_tiling_on_sc=False)


def _wid():
    return lax.axis_index("s") * NC + lax.axis_index("c")


def _deg_body(src_hbm, type_hbm, degp_hbm, src_v, type_v, deg_v):
    wid = _wid()
    base = wid * CHUNK
    pltpu.sync_copy(src_hbm.at[pl.ds(base, CHUNK)], src_v)
    pltpu.sync_copy(type_hbm.at[pl.ds(base, CHUNK)], type_v)

    zero = jnp.zeros((LANES,), jnp.float32)

    def zfill(i, _):
        deg_v[pl.ds(i * LANES, LANES)] = zero
        return 0

    lax.fori_loop(0, (N * NUM_REL + LANES - 1) // LANES, zfill, 0)

    ones = jnp.ones((LANES,), jnp.float32)

    def step(i, _):
        s = src_v[pl.ds(i * LANES, LANES)]
        t = type_v[pl.ds(i * LANES, LANES)]
        plsc.addupdate_scatter(deg_v, [s * NUM_REL + t], ones)
        return 0

    lax.fori_loop(0, VPC, step, 0)
    pltpu.sync_copy(deg_v, degp_hbm.at[wid])


def _deg_call(src, etype):
    k = pl.kernel(
        _deg_body,
        out_type=jax.ShapeDtypeStruct((NW, N * NUM_REL), jnp.float32),
        mesh=_mesh(),
        compiler_params=_SC_PARAMS,
        scratch_types=[
            pltpu.VMEM((CHUNK,), jnp.int32),
            pltpu.VMEM((CHUNK,), jnp.int32),
            pltpu.VMEM((N * NUM_REL,), jnp.float32),
        ],
    )
    return k(src, etype)


def _bucket_body(src_hbm, dst_hbm, type_hbm, bsrc_hbm, bdst_hbm, cnt_hbm,
                 src_v, dst_v, type_v, bsrc_v, bdst_v, cnt_v):
    wid = _wid()
    base = wid * CHUNK
    pltpu.sync_copy(src_hbm.at[pl.ds(base, CHUNK)], src_v)
    pltpu.sync_copy(dst_hbm.at[pl.ds(base, CHUNK)], dst_v)
    pltpu.sync_copy(type_hbm.at[pl.ds(base, CHUNK)], type_v)

    trash = jnp.full((LANES,), TRASH, jnp.int32)
    zero = jnp.zeros((LANES,), jnp.int32)

    def prefill(i, _):
        for r in range(NUM_REL):
            bsrc_v[pl.ds(r * CHUNK + i * LANES, LANES)] = trash
            bdst_v[pl.ds(r * CHUNK + i * LANES, LANES)] = zero
        return 0

    lax.fori_loop(0, VPC, prefill, 0)

    one = jnp.ones((LANES,), jnp.int32)

    def step(i, offs):
        s = src_v[pl.ds(i * LANES, LANES)]
        d = dst_v[pl.ds(i * LANES, LANES)]
        t = type_v[pl.ds(i * LANES, LANES)]
        new = []
        for r in range(NUM_REL):
            m = t == r
            c = plsc.cumsum(jnp.where(m, one, zero))
            pos = offs[r] + c - 1 + r * CHUNK
            plsc.store_scatter(bsrc_v, [pos], s, mask=m)
            plsc.store_scatter(bdst_v, [pos], d, mask=m)
            new.append(offs[r] + plsc.all_reduce_population_count(m))
        return tuple(new)

    offs = lax.fori_loop(0, VPC, step,
                         tuple(jnp.zeros((LANES,), jnp.int32)
                               for _ in range(NUM_REL)))
    for r in range(NUM_REL):
        cnt_v[pl.ds(r * LANES, LANES)] = offs[r]
        pltpu.sync_copy(bsrc_v.at[pl.ds(r * CHUNK, CHUNK)], bsrc_hbm.at[r, wid])
        pltpu.sync_copy(bdst_v.at[pl.ds(r * CHUNK, CHUNK)], bdst_hbm.at[r, wid])
    pltpu.sync_copy(cnt_v, cnt_hbm.at[wid])


def _bucket_call(src, dst, etype):
    k = pl.kernel(
        _bucket_body,
        out_type=(
            jax.ShapeDtypeStruct((NUM_REL, NW, CHUNK), jnp.int32),
            jax.ShapeDtypeStruct((NUM_REL, NW, CHUNK), jnp.int32),
            jax.ShapeDtypeStruct((NW, NUM_REL * LANES), jnp.int32),
        ),
        mesh=_mesh(),
        compiler_params=_SC_PARAMS,
        scratch_types=[
            pltpu.VMEM((CHUNK,), jnp.int32),
            pltpu.VMEM((CHUNK,), jnp.int32),
            pltpu.VMEM((CHUNK,), jnp.int32),
            pltpu.VMEM((NUM_REL * CHUNK,), jnp.int32),
            pltpu.VMEM((NUM_REL * CHUNK,), jnp.int32),
            pltpu.VMEM((NUM_REL * LANES,), jnp.int32),
        ],
    )
    return k(src, dst, etype)


def _segsum_body(h_hbm, bsrc_hbm, bdst_hbm, cnt_hbm, out_hbm,
                 zeros_v, idx_s, idx_d, rows, cnt_v, accum, sem):
    cid = lax.axis_index("c")
    sid = lax.axis_index("s")
    wid = sid * NC + cid

    zero = jnp.zeros((LANES,), jnp.float32)

    def zfill(i, _):
        for j in range(D // LANES):
            zeros_v[i, pl.ds(j * LANES, LANES)] = zero
        return 0

    lax.fori_loop(0, BLK, zfill, 0)
    for k in range(SLAB // BLK):
        pltpu.sync_copy(zeros_v, accum.at[pl.ds(sid * SLAB + k * BLK, BLK)])
    plsc.subcore_barrier()

    pltpu.sync_copy(cnt_hbm.at[wid], cnt_v)
    n = jnp.max(cnt_v[...])
    nblk = (n + BLK - 1) >> 7

    def blk(b, _):
        pltpu.sync_copy(bdst_hbm.at[wid, pl.ds(b * BLK, BLK)], idx_d)
        pltpu.sync_copy(bsrc_hbm.at[wid, pl.ds(b * BLK, BLK)], idx_s)
        pltpu.async_copy(h_hbm.at[idx_d], rows, sem).wait()
        pltpu.sync_copy(rows, accum.at[idx_s], add=True)
        return 0

    lax.fori_loop(0, nblk, blk, 0)
    plsc.subcore_barrier()
    for k in range(SLAB // BLK):
        sl = pl.ds(sid * SLAB + k * BLK, BLK)
        pltpu.sync_copy(accum.at[sl], out_hbm.at[cid, sl])


def _segsum_call(h, bsrc_r, bdst_r, cnt_r):
    k = pl.kernel(
        _segsum_body,
        out_type=jax.ShapeDtypeStruct((NC, NPAD, D), jnp.float32),
        mesh=_mesh(),
        compiler_params=_SC_PARAMS,
        scratch_types=[
            pltpu.VMEM((BLK, D), jnp.float32),
            pltpu.VMEM((BLK,), jnp.int32),
            pltpu.VMEM((BLK,), jnp.int32),
            pltpu.VMEM((BLK, D), jnp.float32),
            pltpu.VMEM((LANES,), jnp.int32),
            pltpu.VMEM_SHARED((NPAD, D), jnp.float32),
            pltpu.SemaphoreType.DMA,
        ],
    )
    return k(h, bsrc_r, bdst_r, cnt_r)


def _dense1_body(x_ref, sp0, sp1, sp2, degp,
                 wr0, wt0, b0, wr1, wt1, b1, wr2, wt2, b2,
                 h0, h1, h2):
    deg = jnp.sum(degp[...], axis=0)
    inv = 1.0 / jnp.maximum(deg, 1.0)
    x = x_ref[...]
    for sp, wr, wt, bb, out, rel in ((sp0, wr0, wt0, b0, h0, METAPATHS[0][0]),
                                     (sp1, wr1, wt1, b1, h1, METAPATHS[1][0]),
                                     (sp2, wr2, wt2, b2, h2, METAPATHS[2][0])):
        s = sp[0] + sp[1]
        agg = s * inv[:, rel:rel + 1]
        out[...] = jnp.maximum(
            jnp.dot(agg, wr[...], preferred_element_type=jnp.float32)
            + jnp.dot(x, wt[...], preferred_element_type=jnp.float32)
            + bb[...], 0.0)


def _dense1_call(x, sp, degp, w):
    row = pl.BlockSpec((BROWS, D), lambda i: (i, 0))
    par = pl.BlockSpec((NC, BROWS, D), lambda i: (0, i, 0))
    degs = pl.BlockSpec((NW, BROWS, NUM_REL), lambda i: (0, i, 0))
    mat = pl.BlockSpec((D, H), lambda i: (0, 0))
    vec = pl.BlockSpec((1, H), lambda i: (0, 0))
    return pl.pallas_call(
        _dense1_body,
        grid=(GRID,),
        in_specs=[row, par, par, par, degs] + [mat, mat, vec] * 3,
        out_specs=[row, row, row],
        out_shape=[jax.ShapeDtypeStruct((N, H), jnp.float32)] * 3,
    )(x, sp[0], sp[1], sp[2], degp,
      w[0][0], w[0][1], w[0][2],
      w[1][0], w[1][1], w[1][2],
      w[2][0], w[2][1], w[2][2])


def _dense2_body(h0r, h1r, h2r, tp0, tp1, tp2, degp,
                 wr0, wt0, b0, wr1, wt1, b1, wr2, wt2, b2,
                 f10, f11, f12, f1b, w2p, b2p, out):
    deg = jnp.sum(degp[...], axis=0)
    inv = 1.0 / jnp.maximum(deg, 1.0)
    g = []
    for hr, tp, wr, wt, bb, rel in ((h0r, tp0, wr0, wt0, b0, METAPATHS[0][1]),
                                    (h1r, tp1, wr1, wt1, b1, METAPATHS[1][1]),
                                    (h2r, tp2, wr2, wt2, b2, METAPATHS[2][1])):
        t = tp[0] + tp[1]
        agg = t * inv[:, rel:rel + 1]
        g.append(jnp.maximum(
            jnp.dot(agg, wr[...], preferred_element_type=jnp.float32)
            + jnp.dot(hr[...], wt[...], preferred_element_type=jnp.float32)
            + bb[...], 0.0))
    z = jnp.maximum(
        jnp.dot(g[0], f10[...], preferred_element_type=jnp.float32)
        + jnp.dot(g[1], f11[...], preferred_element_type=jnp.float32)
        + jnp.dot(g[2], f12[...], preferred_element_type=jnp.float32)
        + f1b[...], 0.0)
    logits = jnp.dot(z, w2p[...], preferred_element_type=jnp.float32) + b2p[...]
    m = jnp.max(logits, axis=1, keepdims=True)
    lse = m + jnp.log(jnp.sum(jnp.exp(logits - m), axis=1, keepdims=True))
    out[...] = (logits - lse)[:, :NCLS]


def _dense2_call(h, tp, degp, w, f1, f1b, w2p, b2p):
    row = pl.BlockSpec((BROWS, D), lambda i: (i, 0))
    par = pl.BlockSpec((NC, BROWS, D), lambda i: (0, i, 0))
    degs = pl.BlockSpec((NW, BROWS, NUM_REL), lambda i: (0, i, 0))
    mat = pl.BlockSpec((D, H), lambda i: (0, 0))
    vec = pl.BlockSpec((1, H), lambda i: (0, 0))
    outs = pl.BlockSpec((BROWS, NCLS), lambda i: (i, 0))
    return pl.pallas_call(
        _dense2_body,
        grid=(GRID,),
        in_specs=[row, row, row, par, par, par, degs]
                 + [mat, mat, vec] * 3 + [mat, mat, mat, vec, mat, vec],
        out_specs=outs,
        out_shape=jax.ShapeDtypeStruct((N, NCLS), jnp.float32),
    )(h[0], h[1], h[2], tp[0], tp[1], tp[2], degp,
      w[0][0], w[0][1], w[0][2],
      w[1][0], w[1][1], w[1][2],
      w[2][0], w[2][1], w[2][2],
      f1[0], f1[1], f1[2], f1b, w2p, b2p)


def kernel(x, edge_index, edge_type,
           Wrel_0_0, Wroot_0_0, b_0_0, Wrel_0_1, Wroot_0_1, b_0_1,
           Wrel_1_0, Wroot_1_0, b_1_0, Wrel_1_1, Wroot_1_1, b_1_1,
           Wrel_2_0, Wroot_2_0, b_2_0, Wrel_2_1, Wroot_2_1, b_2_1,
           fc1_W, fc1_b, fc2_W, fc2_b):
    src = edge_index[0]
    dst = edge_index[1]

    degp = _deg_call(src, edge_type)
    degp = degp.reshape(NW, N, NUM_REL)
    degp = jnp.concatenate(
        [degp, jnp.zeros((NW, NPAD - N, NUM_REL), jnp.float32)], axis=1)

    bsrc, bdst, cnt = _bucket_call(src, dst, edge_type)
    cnt_r = [cnt[:, r * LANES:(r + 1) * LANES] for r in range(NUM_REL)]

    wrel = ((Wrel_0_0, Wrel_0_1), (Wrel_1_0, Wrel_1_1), (Wrel_2_0, Wrel_2_1))
    wroot = ((Wroot_0_0, Wroot_0_1), (Wroot_1_0, Wroot_1_1),
             (Wroot_2_0, Wroot_2_1))
    bias = ((b_0_0, b_0_1), (b_1_0, b_1_1), (b_2_0, b_2_1))

    sp = []
    w1 = []
    for i, mp in enumerate(METAPATHS):
        r = mp[0]
        sp.append(_segsum_call(x, bsrc[r], bdst[r], cnt_r[r]))
        w1.append((wrel[i][0][r], wroot[i][0], bias[i][0].reshape(1, H)))
    h = _dense1_call(x, sp, degp, w1)

    tp = []
    w2 = []
    for i, mp in enumerate(METAPATHS):
        r = mp[1]
        tp.append(_segsum_call(h[i], bsrc[r], bdst[r], cnt_r[r]))
        w2.append((wrel[i][1][r], wroot[i][1], bias[i][1].reshape(1, H)))

    f1 = [fc1_W[i * H:(i + 1) * H] for i in range(3)]
    w2pad = jnp.zeros((H, H), jnp.float32).at[:, :NCLS].set(fc2_W)
    b2pad = jnp.full((1, H), -1e30, jnp.float32).at[0, :NCLS].set(fc2_b)

    return _dense2_call(h, tp, degp, w2, f1, fc1_b.reshape(1, H), w2pad, b2pad)
